# baseline (speedup 1.0000x reference)
import functools

import jax
import jax.numpy as jnp
from jax import lax
from jax.experimental import pallas as pl
from jax.experimental.pallas import tpu as pltpu
from jax.experimental.pallas import tpu_sc as plsc

NUM_VIDEOS = 100000
NUM_FREQ = 8
DIM = 32
ROW = DIM * (2 * NUM_FREQ + 1)
BATCH = 16384
HIST = 50
N = BATCH * HIST

NC = 2
NS = 16
NW = NC * NS
PW = N // NW
K = 128
CHUNKS = PW // K


def _basis_tc(times_flat):
    BL = 2048
    grid = N // BL

    def body(t_ref, o_ref):
        t = t_ref[...]
        ri = lax.broadcasted_iota(jnp.int32, (16, BL), 0)
        f = jnp.exp2(jnp.where(ri < 8, ri, ri - 8).astype(jnp.float32)) * jnp.pi
        ph = t * f
        o_ref[...] = jnp.where(ri < 8, jnp.sin(ph), jnp.cos(ph))

    return pl.pallas_call(
        body,
        grid=(grid,),
        in_specs=[pl.BlockSpec((1, BL), lambda i: (0, i))],
        out_specs=pl.BlockSpec((16, BL), lambda i: (0, i)),
        out_shape=jax.ShapeDtypeStruct((16, N), jnp.float32),
    )(times_flat.reshape(1, N))


_GDN = lax.GatherDimensionNumbers(
    offset_dims=(), collapsed_slice_dims=(0,), start_index_map=(0,)
)


def _bcast(vec, j):
    idx = jnp.full((16, 1), j, jnp.int32)
    return lax.gather(
        vec, idx, _GDN, (1,), mode=lax.GatherScatterMode.PROMISE_IN_BOUNDS
    )


def _sc_embed(weights2d, vids, basis):
    mesh = plsc.VectorSubcoreMesh(
        core_axis_name="c", subcore_axis_name="s", num_cores=NC, num_subcores=NS
    )

    @functools.partial(
        pl.kernel,
        mesh=mesh,
        compiler_params=pltpu.CompilerParams(
            use_tc_tiling_on_sc=False, needs_layout_passes=False
        ),
        out_type=jax.ShapeDtypeStruct((N, DIM), jnp.float32),
        scratch_types=[
            pltpu.VMEM((K,), jnp.int32),
            pltpu.VMEM((K, ROW), jnp.float32),
            pltpu.VMEM((K, 16), jnp.float32),
            pltpu.VMEM((K, DIM), jnp.float32),
            pltpu.SemaphoreType.DMA,
        ],
    )
    def k(w_hbm, v_hbm, b_hbm, out_hbm, idx_v, rows_v, b_v, out_v, sem):
        wid = lax.axis_index("s") * NC + lax.axis_index("c")
        base_n = wid * PW
        d17a = lax.iota(jnp.int32, 16) * 17
        d17b = d17a + 16 * 17

        def chunk(g, _):
            n0 = base_n + g * K
            pltpu.sync_copy(v_hbm.at[pl.ds(n0, K)], idx_v)
            pltpu.async_copy(w_hbm.at[idx_v], rows_v, sem).wait()
            pltpu.sync_copy(b_hbm.at[pl.ds(n0, K)], b_v)

            def grp(gi, _):
                for u in range(16):
                    s = gi * 16 + u
                    row = rows_v.at[s]
                    bb = [_bcast(b_v[s, :], j) for j in range(16)]
                    for half, dbase in ((0, d17a), (1, d17b)):
                        w = [
                            plsc.load_gather(row, [dbase + j])
                            for j in range(17)
                        ]
                        accs = [w[0], w[1] * bb[0], w[2] * bb[1], w[3] * bb[2]]
                        for j in range(4, 17):
                            accs[j % 4] = accs[j % 4] + w[j] * bb[j - 1]
                        acc = (accs[0] + accs[1]) + (accs[2] + accs[3])
                        out_v[s, pl.ds(half * 16, 16)] = acc
                return 0

            lax.fori_loop(0, K // 16, grp, 0)

            pltpu.sync_copy(out_v, out_hbm.at[pl.ds(n0, K)])
            return 0

        lax.fori_loop(0, CHUNKS, chunk, 0)

    return k(weights2d, vids, basis)


def kernel(times, video_ids, weights):
    vids = video_ids.reshape(-1).astype(jnp.int32)
    w2 = weights.reshape(NUM_VIDEOS, ROW)
    basis = _basis_tc(times).T
    out = _sc_embed(w2, vids, basis)
    return out.reshape(BATCH, HIST * DIM)


# speedup vs baseline: 1.3154x; 1.3154x over previous
import functools

import jax
import jax.numpy as jnp
from jax import lax
from jax.experimental import pallas as pl
from jax.experimental.pallas import tpu as pltpu
from jax.experimental.pallas import tpu_sc as plsc

NUM_VIDEOS = 100000
NUM_FREQ = 8
DIM = 32
ROW = DIM * (2 * NUM_FREQ + 1)
BATCH = 16384
HIST = 50
N = BATCH * HIST

NC = 2
NS = 16
NW = NC * NS
PW = N // NW
K = 80
CHUNKS = PW // K


def _basis_tc(times_flat):
    BL = 2048
    grid = N // BL

    def body(t_ref, o_ref):
        t = t_ref[...]
        ri = lax.broadcasted_iota(jnp.int32, (16, BL), 0)
        f = jnp.exp2(jnp.where(ri < 8, ri, ri - 8).astype(jnp.float32)) * jnp.pi
        ph = t * f
        o_ref[...] = jnp.where(ri < 8, jnp.sin(ph), jnp.cos(ph))

    return pl.pallas_call(
        body,
        grid=(grid,),
        in_specs=[pl.BlockSpec((1, BL), lambda i: (0, i))],
        out_specs=pl.BlockSpec((16, BL), lambda i: (0, i)),
        out_shape=jax.ShapeDtypeStruct((16, N), jnp.float32),
    )(times_flat.reshape(1, N))


_GDN = lax.GatherDimensionNumbers(
    offset_dims=(), collapsed_slice_dims=(0,), start_index_map=(0,)
)


def _bcast(vec, j):
    idx = jnp.full((16, 1), j, jnp.int32)
    return lax.gather(
        vec, idx, _GDN, (1,), mode=lax.GatherScatterMode.PROMISE_IN_BOUNDS
    )


def _sc_embed(weights2d, vids, basis):
    mesh = plsc.VectorSubcoreMesh(
        core_axis_name="c", subcore_axis_name="s", num_cores=NC, num_subcores=NS
    )

    @functools.partial(
        pl.kernel,
        mesh=mesh,
        compiler_params=pltpu.CompilerParams(
            use_tc_tiling_on_sc=False, needs_layout_passes=False
        ),
        out_type=jax.ShapeDtypeStruct((N, DIM), jnp.float32),
        scratch_types=[
            pltpu.VMEM((2, K), jnp.int32),
            pltpu.VMEM((2, K, ROW), jnp.float32),
            pltpu.VMEM((2, K, 16), jnp.float32),
            pltpu.VMEM((2, K, DIM), jnp.float32),
            pltpu.SemaphoreType.DMA((2,)),
            pltpu.SemaphoreType.DMA((2,)),
            pltpu.SemaphoreType.DMA((2,)),
            pltpu.SemaphoreType.DMA((2,)),
        ],
    )
    def k(w_hbm, v_hbm, b_hbm, out_hbm, idx_v, rows_v, b_v, out_v,
          sem_r, sem_b, sem_i, sem_o):
        wid = lax.axis_index("s") * NC + lax.axis_index("c")
        base_n = wid * PW
        d17a = lax.iota(jnp.int32, 16) * 17
        d17b = d17a + 16 * 17

        def fetch(g, buf):
            n0 = base_n + g * K
            pltpu.async_copy(
                w_hbm.at[idx_v.at[buf]], rows_v.at[buf], sem_r.at[buf]
            )
            pltpu.async_copy(
                b_hbm.at[pl.ds(n0, K)], b_v.at[buf], sem_b.at[buf]
            )

        pltpu.sync_copy(v_hbm.at[pl.ds(base_n, K)], idx_v.at[0])
        pltpu.async_copy(
            v_hbm.at[pl.ds(base_n + K, K)], idx_v.at[1], sem_i.at[1]
        )
        fetch(0, 0)

        def chunk(g, _):
            buf = lax.rem(g, 2)
            nxt = lax.rem(g + 1, 2)
            n0 = base_n + g * K

            pltpu.make_async_copy(
                w_hbm.at[idx_v.at[buf]], rows_v.at[buf], sem_r.at[buf]
            ).wait()
            pltpu.make_async_copy(
                b_hbm.at[pl.ds(n0, K)], b_v.at[buf], sem_b.at[buf]
            ).wait()

            @pl.when(g + 2 < CHUNKS)
            def _():
                pltpu.async_copy(
                    v_hbm.at[pl.ds(n0 + 2 * K, K)], idx_v.at[buf],
                    sem_i.at[buf],
                )

            @pl.when(g + 1 < CHUNKS)
            def _():
                pltpu.make_async_copy(
                    v_hbm.at[pl.ds(n0 + K, K)], idx_v.at[nxt], sem_i.at[nxt]
                ).wait()
                fetch(g + 1, nxt)

            @pl.when(g >= 2)
            def _():
                pltpu.make_async_copy(
                    out_v.at[buf], out_hbm.at[pl.ds(n0 - 2 * K, K)],
                    sem_o.at[buf],
                ).wait()

            def grp(gi, _):
                for u in range(16):
                    s = gi * 16 + u
                    row = rows_v.at[buf].at[s]
                    bb = [_bcast(b_v[buf, s, :], j) for j in range(16)]
                    for half, dbase in ((0, d17a), (1, d17b)):
                        w = [
                            plsc.load_gather(row, [dbase + j])
                            for j in range(17)
                        ]
                        accs = [w[0], w[1] * bb[0], w[2] * bb[1], w[3] * bb[2]]
                        for j in range(4, 17):
                            accs[j % 4] = accs[j % 4] + w[j] * bb[j - 1]
                        acc = (accs[0] + accs[1]) + (accs[2] + accs[3])
                        out_v[buf, s, pl.ds(half * 16, 16)] = acc
                return 0

            lax.fori_loop(0, K // 16, grp, 0)

            pltpu.async_copy(
                out_v.at[buf], out_hbm.at[pl.ds(n0, K)], sem_o.at[buf]
            )
            return 0

        lax.fori_loop(0, CHUNKS, chunk, 0)

        for g in (CHUNKS - 2, CHUNKS - 1):
            buf = g % 2
            pltpu.make_async_copy(
                out_v.at[buf], out_hbm.at[pl.ds(base_n + g * K, K)],
                sem_o.at[buf],
            ).wait()

    return k(weights2d, vids, basis)


def kernel(times, video_ids, weights):
    vids = video_ids.reshape(-1).astype(jnp.int32)
    w2 = weights.reshape(NUM_VIDEOS, ROW)
    basis = _basis_tc(times).T
    out = _sc_embed(w2, vids, basis)
    return out.reshape(BATCH, HIST * DIM)
